# Initial kernel scaffold; baseline (speedup 1.0000x reference)
import jax
import jax.numpy as jnp
from jax.experimental import pallas as pl


def kernel(Z, element_embedding, config_weight, electron_config):
    raise NotImplementedError("write your pallas kernel here")


# speedup vs baseline: 2.8405x; 2.8405x over previous
import functools

import jax
import jax.numpy as jnp
from jax import lax
from jax.experimental import pallas as pl
from jax.experimental.pallas import tpu as pltpu
from jax.experimental.pallas import tpu_sc as plsc

_NUM_FEATURES = 128
_ZMAX = 87

_NUM_CORES = 2
_NUM_SUBCORES = 16
_NW = _NUM_CORES * _NUM_SUBCORES

_CHUNK = 128


def _table_body(emb_ref, ec_ref, cw_ref, out_ref):
    out_ref[...] = emb_ref[...] + lax.dot_general(
        ec_ref[...], cw_ref[...],
        dimension_numbers=(((1,), (1,)), ((), ())),
        preferred_element_type=jnp.float32,
    )


def _build_table(element_embedding, config_weight, electron_config):
    return pl.pallas_call(
        _table_body,
        out_shape=jax.ShapeDtypeStruct((_ZMAX, _NUM_FEATURES), jnp.float32),
    )(element_embedding, electron_config, config_weight)


def _sc_gather(table, z_flat):
    n = z_flat.shape[0]
    b_per_w = n // _NW
    n_chunks = b_per_w // _CHUNK
    zr = z_flat.reshape(_NW, n_chunks, _CHUNK)
    mesh = plsc.VectorSubcoreMesh(core_axis_name="c", subcore_axis_name="s")

    @functools.partial(
        pl.kernel,
        mesh=mesh,
        out_type=jax.ShapeDtypeStruct((n, _NUM_FEATURES), jnp.float32),
        scratch_types=[
            pltpu.VMEM((n_chunks, _CHUNK), jnp.int32),
            pltpu.VMEM((_CHUNK, _NUM_FEATURES), jnp.float32),
            pltpu.VMEM((_CHUNK, _NUM_FEATURES), jnp.float32),
            pltpu.SemaphoreType.DMA,
            pltpu.SemaphoreType.DMA,
            pltpu.SemaphoreType.DMA,
        ],
    )
    def k(table_hbm, idx_hbm, out_hbm, idx_v, buf0, buf1, gsem, osem0, osem1):
        wid = lax.axis_index("s") * _NUM_CORES + lax.axis_index("c")
        base = wid * b_per_w
        pltpu.sync_copy(idx_hbm.at[wid], idx_v)

        def body(j, _):
            def step(p, buf, osem):
                @pl.when(j % 2 == p)
                def _():
                    @pl.when(j >= 2)
                    def _():
                        pltpu.make_async_copy(
                            buf, out_hbm.at[pl.ds(0, _CHUNK)], osem
                        ).wait()

                    pltpu.async_copy(table_hbm.at[idx_v.at[j]], buf, gsem).wait()
                    pltpu.async_copy(
                        buf, out_hbm.at[pl.ds(base + j * _CHUNK, _CHUNK)], osem
                    )

            step(0, buf0, osem0)
            step(1, buf1, osem1)
            return 0

        lax.fori_loop(0, n_chunks, body, 0)
        pltpu.make_async_copy(buf0, out_hbm.at[pl.ds(0, _CHUNK)], osem0).wait()
        pltpu.make_async_copy(buf1, out_hbm.at[pl.ds(0, _CHUNK)], osem1).wait()

    return k(table, zr)


def kernel(Z, element_embedding, config_weight, electron_config):
    table = _build_table(element_embedding, config_weight, electron_config)
    out = _sc_gather(table, Z.reshape(-1))
    return out.reshape(Z.shape + (_NUM_FEATURES,))


# speedup vs baseline: 11.6522x; 4.1021x over previous
import functools

import jax
import jax.numpy as jnp
from jax import lax
from jax.experimental import pallas as pl
from jax.experimental.pallas import tpu as pltpu
from jax.experimental.pallas import tpu_sc as plsc

_NUM_FEATURES = 128
_ZMAX = 87

_NUM_CORES = 2
_NUM_SUBCORES = 16
_NW = _NUM_CORES * _NUM_SUBCORES

_CHUNK = 128


def _table_body(emb_ref, ec_ref, cw_ref, out_ref):
    out_ref[...] = emb_ref[...] + lax.dot_general(
        ec_ref[...], cw_ref[...],
        dimension_numbers=(((1,), (1,)), ((), ())),
        preferred_element_type=jnp.float32,
    )


def _build_table(element_embedding, config_weight, electron_config):
    return pl.pallas_call(
        _table_body,
        out_shape=jax.ShapeDtypeStruct((_ZMAX, _NUM_FEATURES), jnp.float32),
    )(element_embedding, electron_config, config_weight)


def _sc_gather(table, z_flat):
    n = z_flat.shape[0]
    b_per_w = n // _NW
    n_chunks = b_per_w // _CHUNK
    zr = z_flat.reshape(_NW, n_chunks, _CHUNK)
    mesh = plsc.VectorSubcoreMesh(core_axis_name="c", subcore_axis_name="s")

    @functools.partial(
        pl.kernel,
        mesh=mesh,
        out_type=jax.ShapeDtypeStruct((n, _NUM_FEATURES), jnp.float32),
        scratch_types=[
            pltpu.VMEM_SHARED((_ZMAX, _NUM_FEATURES), jnp.float32),
            pltpu.VMEM((n_chunks, _CHUNK), jnp.int32),
            pltpu.VMEM((_CHUNK, _NUM_FEATURES), jnp.float32),
            pltpu.VMEM((_CHUNK, _NUM_FEATURES), jnp.float32),
            pltpu.SemaphoreType.DMA,
            pltpu.SemaphoreType.DMA,
            pltpu.SemaphoreType.DMA,
        ],
    )
    def k(table_hbm, idx_hbm, out_hbm, table_sp, idx_v, buf0, buf1, gsem,
          osem0, osem1):
        wid = lax.axis_index("s") * _NUM_CORES + lax.axis_index("c")
        base = wid * b_per_w

        @pl.when(lax.axis_index("s") == 0)
        def _():
            pltpu.sync_copy(table_hbm, table_sp)

        pltpu.sync_copy(idx_hbm.at[wid], idx_v)
        plsc.subcore_barrier()

        def body(j, _):
            def step(p, buf, osem):
                @pl.when(j % 2 == p)
                def _():
                    @pl.when(j >= 2)
                    def _():
                        pltpu.make_async_copy(
                            buf, out_hbm.at[pl.ds(0, _CHUNK)], osem
                        ).wait()

                    pltpu.async_copy(table_sp.at[idx_v.at[j]], buf, gsem).wait()
                    pltpu.async_copy(
                        buf, out_hbm.at[pl.ds(base + j * _CHUNK, _CHUNK)], osem
                    )

            step(0, buf0, osem0)
            step(1, buf1, osem1)
            return 0

        lax.fori_loop(0, n_chunks, body, 0)
        pltpu.make_async_copy(buf0, out_hbm.at[pl.ds(0, _CHUNK)], osem0).wait()
        pltpu.make_async_copy(buf1, out_hbm.at[pl.ds(0, _CHUNK)], osem1).wait()

    return k(table, zr)


def kernel(Z, element_embedding, config_weight, electron_config):
    table = _build_table(element_embedding, config_weight, electron_config)
    out = _sc_gather(table, Z.reshape(-1))
    return out.reshape(Z.shape + (_NUM_FEATURES,))


# speedup vs baseline: 12.2918x; 1.0549x over previous
import functools

import jax
import jax.numpy as jnp
from jax import lax
from jax.experimental import pallas as pl
from jax.experimental.pallas import tpu as pltpu
from jax.experimental.pallas import tpu_sc as plsc

_NUM_FEATURES = 128
_ZMAX = 87

_NUM_CORES = 2
_NUM_SUBCORES = 16
_NW = _NUM_CORES * _NUM_SUBCORES

_CHUNK = 128
_NBUF = 4


def _table_body(emb_ref, ec_ref, cw_ref, out_ref):
    out_ref[...] = emb_ref[...] + lax.dot_general(
        ec_ref[...], cw_ref[...],
        dimension_numbers=(((1,), (1,)), ((), ())),
        preferred_element_type=jnp.float32,
    )


def _build_table(element_embedding, config_weight, electron_config):
    return pl.pallas_call(
        _table_body,
        out_shape=jax.ShapeDtypeStruct((_ZMAX, _NUM_FEATURES), jnp.float32),
    )(element_embedding, electron_config, config_weight)


def _sc_gather(table, z_flat):
    n = z_flat.shape[0]
    b_per_w = n // _NW
    n_chunks = b_per_w // _CHUNK
    zr = z_flat.reshape(_NW, n_chunks, _CHUNK)
    mesh = plsc.VectorSubcoreMesh(core_axis_name="c", subcore_axis_name="s")

    @functools.partial(
        pl.kernel,
        mesh=mesh,
        out_type=jax.ShapeDtypeStruct((n, _NUM_FEATURES), jnp.float32),
        scratch_types=[
            pltpu.VMEM_SHARED((_ZMAX, _NUM_FEATURES), jnp.float32),
            pltpu.VMEM((n_chunks, _CHUNK), jnp.int32),
        ]
        + [pltpu.VMEM((_CHUNK, _NUM_FEATURES), jnp.float32)] * _NBUF
        + [pltpu.SemaphoreType.DMA] * (2 * _NBUF),
    )
    def k(table_hbm, idx_hbm, out_hbm, table_sp, idx_v, *bs):
        bufs, gsems, osems = bs[:_NBUF], bs[_NBUF:2 * _NBUF], bs[2 * _NBUF:]
        wid = lax.axis_index("s") * _NUM_CORES + lax.axis_index("c")
        base = wid * b_per_w

        def start_gather(j, p):
            pltpu.async_copy(table_sp.at[idx_v.at[j]], bufs[p], gsems[p])

        def wait_gather(p):
            pltpu.make_async_copy(
                out_hbm.at[pl.ds(0, _CHUNK)], bufs[p], gsems[p]
            ).wait()

        def wait_scatter(p):
            pltpu.make_async_copy(
                bufs[p], out_hbm.at[pl.ds(0, _CHUNK)], osems[p]
            ).wait()

        @pl.when(lax.axis_index("s") == 0)
        def _():
            pltpu.sync_copy(table_hbm, table_sp)

        pltpu.sync_copy(idx_hbm.at[wid], idx_v)
        plsc.subcore_barrier()

        start_gather(0, 0)
        start_gather(1, 1)

        def body(j, _):
            for p in range(_NBUF):
                @pl.when(j % _NBUF == p)
                def _(p=p):
                    wait_gather(p)
                    pltpu.async_copy(
                        bufs[p],
                        out_hbm.at[pl.ds(base + j * _CHUNK, _CHUNK)],
                        osems[p],
                    )

            @pl.when(j + 2 < n_chunks)
            def _():
                for q in range(_NBUF):
                    @pl.when((j + 2) % _NBUF == q)
                    def _(q=q):
                        @pl.when(j >= 2)
                        def _():
                            wait_scatter(q)

                        start_gather(j + 2, q)

            return 0

        lax.fori_loop(0, n_chunks, body, 0)
        for p in range(_NBUF):
            wait_scatter(p)

    return k(table, zr)


def kernel(Z, element_embedding, config_weight, electron_config):
    table = _build_table(element_embedding, config_weight, electron_config)
    out = _sc_gather(table, Z.reshape(-1))
    return out.reshape(Z.shape + (_NUM_FEATURES,))


# speedup vs baseline: 12.3543x; 1.0051x over previous
import functools

import jax
import jax.numpy as jnp
from jax import lax
from jax.experimental import pallas as pl
from jax.experimental.pallas import tpu as pltpu
from jax.experimental.pallas import tpu_sc as plsc

_NUM_FEATURES = 128
_ZMAX = 87

_NUM_CORES = 2
_NUM_SUBCORES = 16
_NW = _NUM_CORES * _NUM_SUBCORES

_CHUNK = 128
_NBUF = 6
_LOOK = 3


def _table_body(emb_ref, ec_ref, cw_ref, out_ref):
    out_ref[...] = emb_ref[...] + lax.dot_general(
        ec_ref[...], cw_ref[...],
        dimension_numbers=(((1,), (1,)), ((), ())),
        preferred_element_type=jnp.float32,
    )


def _build_table(element_embedding, config_weight, electron_config):
    return pl.pallas_call(
        _table_body,
        out_shape=jax.ShapeDtypeStruct((_ZMAX, _NUM_FEATURES), jnp.float32),
    )(element_embedding, electron_config, config_weight)


def _sc_gather(table, z_flat):
    n = z_flat.shape[0]
    b_per_w = n // _NW
    n_chunks = b_per_w // _CHUNK
    mesh = plsc.VectorSubcoreMesh(core_axis_name="c", subcore_axis_name="s")

    @functools.partial(
        pl.kernel,
        mesh=mesh,
        out_type=jax.ShapeDtypeStruct((n, _NUM_FEATURES), jnp.float32),
        scratch_types=[
            pltpu.VMEM_SHARED((_ZMAX, _NUM_FEATURES), jnp.float32),
            pltpu.VMEM((b_per_w,), jnp.int32),
        ]
        + [pltpu.VMEM((_CHUNK, _NUM_FEATURES), jnp.float32)] * _NBUF
        + [pltpu.SemaphoreType.DMA] * (2 * _NBUF),
    )
    def k(table_hbm, idx_hbm, out_hbm, table_sp, idx_v, *bs):
        bufs, gsems, osems = bs[:_NBUF], bs[_NBUF:2 * _NBUF], bs[2 * _NBUF:]
        wid = lax.axis_index("s") * _NUM_CORES + lax.axis_index("c")
        base = wid * b_per_w

        def start_gather(j, p):
            pltpu.async_copy(
                table_sp.at[idx_v.at[pl.ds(j * _CHUNK, _CHUNK)]],
                bufs[p],
                gsems[p],
            )

        def wait_gather(p):
            pltpu.make_async_copy(
                out_hbm.at[pl.ds(0, _CHUNK)], bufs[p], gsems[p]
            ).wait()

        def wait_scatter(p):
            pltpu.make_async_copy(
                bufs[p], out_hbm.at[pl.ds(0, _CHUNK)], osems[p]
            ).wait()

        @pl.when(lax.axis_index("s") == 0)
        def _():
            pltpu.sync_copy(table_hbm, table_sp)

        pltpu.sync_copy(idx_hbm.at[pl.ds(base, b_per_w)], idx_v)
        plsc.subcore_barrier()

        for j in range(_LOOK):
            start_gather(j, j % _NBUF)

        def body(j, _):
            for p in range(_NBUF):
                @pl.when(j % _NBUF == p)
                def _(p=p):
                    wait_gather(p)
                    pltpu.async_copy(
                        bufs[p],
                        out_hbm.at[pl.ds(base + j * _CHUNK, _CHUNK)],
                        osems[p],
                    )

            @pl.when(j + _LOOK < n_chunks)
            def _():
                for q in range(_NBUF):
                    @pl.when((j + _LOOK) % _NBUF == q)
                    def _(q=q):
                        @pl.when(j + _LOOK >= _NBUF)
                        def _():
                            wait_scatter(q)

                        start_gather(j + _LOOK, q)

            return 0

        lax.fori_loop(0, n_chunks, body, 0)
        for p in range(_NBUF):
            wait_scatter(p)

    return k(table, z_flat)


def kernel(Z, element_embedding, config_weight, electron_config):
    table = _build_table(element_embedding, config_weight, electron_config)
    out = _sc_gather(table, Z.reshape(-1))
    return out.reshape(Z.shape + (_NUM_FEATURES,))
